# Initial kernel scaffold; baseline (speedup 1.0000x reference)
import jax
import jax.numpy as jnp
from jax.experimental import pallas as pl


def kernel(input_seq, table):
    raise NotImplementedError("write your pallas kernel here")


# speedup vs baseline: 6.3377x; 6.3377x over previous
import functools

import jax
import jax.numpy as jnp
from jax import lax
from jax.experimental import pallas as pl
from jax.experimental.pallas import tpu as pltpu
from jax.experimental.pallas import tpu_sc as plsc

EMBED = 128
NC, NS = 2, 16
NW = NC * NS
CHUNK = 128


def kernel(input_seq, table):
    B, L = input_seq.shape
    total = B * L
    b_per_w = total // NW
    n_chunks = b_per_w // CHUNK
    idx = input_seq.reshape(NW, n_chunks, CHUNK).astype(jnp.int32)

    mesh = plsc.VectorSubcoreMesh(core_axis_name="c", subcore_axis_name="s")

    @functools.partial(
        pl.kernel,
        mesh=mesh,
        out_type=jax.ShapeDtypeStruct((NW, b_per_w, EMBED), jnp.float32),
        scratch_types=[
            pltpu.VMEM((n_chunks, CHUNK), jnp.int32),
            pltpu.VMEM((CHUNK, EMBED), jnp.float32),
            pltpu.SemaphoreType.DMA,
        ],
    )
    def emb_kernel(idx_hbm, table_hbm, out_hbm, idx_v, rows_v, sem):
        wid = lax.axis_index("s") * NC + lax.axis_index("c")
        pltpu.sync_copy(idx_hbm.at[wid], idx_v)

        def body(j, carry):
            pltpu.async_copy(table_hbm.at[idx_v.at[j]], rows_v, sem).wait()
            pltpu.sync_copy(rows_v, out_hbm.at[wid, pl.ds(j * CHUNK, CHUNK)])
            return carry

        lax.fori_loop(0, n_chunks, body, 0)

    out = emb_kernel(idx, table)
    return out.reshape(B, L, EMBED)


# speedup vs baseline: 9.1831x; 1.4490x over previous
import functools

import jax
import jax.numpy as jnp
from jax import lax
from jax.experimental import pallas as pl
from jax.experimental.pallas import tpu as pltpu
from jax.experimental.pallas import tpu_sc as plsc

EMBED = 128
NC, NS = 2, 16
NW = NC * NS
CHUNK = 128
NBUF = 4


def kernel(input_seq, table):
    B, L = input_seq.shape
    total = B * L
    b_per_w = total // NW
    n_chunks = b_per_w // CHUNK
    ngroups = n_chunks // NBUF
    idx = input_seq.reshape(NW, n_chunks, CHUNK).astype(jnp.int32)

    mesh = plsc.VectorSubcoreMesh(core_axis_name="c", subcore_axis_name="s")

    @functools.partial(
        pl.kernel,
        mesh=mesh,
        out_type=jax.ShapeDtypeStruct((NW, b_per_w, EMBED), jnp.float32),
        scratch_types=[
            pltpu.VMEM((n_chunks, CHUNK), jnp.int32),
            pltpu.VMEM((NBUF, CHUNK, EMBED), jnp.float32),
        ]
        + [pltpu.SemaphoreType.DMA] * (2 * NBUF),
    )
    def emb_kernel(idx_hbm, table_hbm, out_hbm, idx_v, rows_v, *sems):
        gsems, wsems = sems[:NBUF], sems[NBUF:]
        wid = lax.axis_index("s") * NC + lax.axis_index("c")
        pltpu.sync_copy(idx_hbm.at[wid], idx_v)

        def gather(j, b):
            return pltpu.make_async_copy(
                table_hbm.at[idx_v.at[j]], rows_v.at[b], gsems[b])

        def write(j, b):
            return pltpu.make_async_copy(
                rows_v.at[b], out_hbm.at[wid, pl.ds(j * CHUNK, CHUNK)],
                wsems[b])

        for b in range(NBUF):
            gather(b, b).start()

        def group(g, carry):
            j0 = g * NBUF
            for b in range(NBUF):
                gather(j0 + b, b).wait()
                write(j0 + b, b).start()

            @pl.when(g + 1 < ngroups)
            def _():
                for b in range(NBUF):
                    write(j0 + b, b).wait()
                    gather(j0 + NBUF + b, b).start()

            return carry

        lax.fori_loop(0, ngroups, group, 0)

        for b in range(NBUF):
            write((ngroups - 1) * NBUF + b, b).wait()

    out = emb_kernel(idx, table)
    return out.reshape(B, L, EMBED)
